# Initial kernel scaffold; baseline (speedup 1.0000x reference)
import jax
import jax.numpy as jnp
from jax.experimental import pallas as pl


def kernel(input_seq, W0, W1, W2):
    raise NotImplementedError("write your pallas kernel here")


# speedup vs baseline: 2.4231x; 2.4231x over previous
import functools

import jax
import jax.numpy as jnp
from jax import lax
from jax.experimental import pallas as pl
from jax.experimental.pallas import tpu as pltpu
from jax.experimental.pallas import tpu_sc as plsc

NUM_TABLES = 3
DIM = 128
CHUNK = 128
NBUF = 4


def _make_sc_gather(total_rows: int):
    info = plsc.get_sparse_core_info()
    nw = info.num_cores * info.num_subcores
    rows_per_w = total_rows // nw
    chunks_per_w = rows_per_w // CHUNK
    idx_rows_per_w = rows_per_w // CHUNK

    mesh = plsc.VectorSubcoreMesh(core_axis_name="c", subcore_axis_name="s")

    @functools.partial(
        pl.kernel,
        out_type=jax.ShapeDtypeStruct((NUM_TABLES, total_rows, DIM),
                                      jnp.float32),
        mesh=mesh,
        scratch_types=(
            [pltpu.VMEM((idx_rows_per_w, CHUNK), jnp.int32)]
            + [pltpu.VMEM((CHUNK, DIM), jnp.float32) for _ in range(NBUF)]
            + [pltpu.SemaphoreType.DMA for _ in range(2 * NBUF)]
        ),
    )
    def k(idx_hbm, w0, w1, w2, out_hbm, idx_v, *bufs_and_sems):
        bufs = bufs_and_sems[:NBUF]
        gsem = bufs_and_sems[NBUF:2 * NBUF]
        ssem = bufs_and_sems[2 * NBUF:]
        wid = lax.axis_index("s") * info.num_cores + lax.axis_index("c")
        base = wid * rows_per_w

        pltpu.sync_copy(idx_hbm.at[pl.ds(wid * idx_rows_per_w,
                                         idx_rows_per_w)], idx_v)

        tables = (w0, w1, w2)
        n_chunks = NUM_TABLES * chunks_per_w

        def fire_gather(c):
            t, j = divmod(c, chunks_per_w)
            b = c % NBUF
            return pltpu.async_copy(tables[t].at[idx_v.at[j]], bufs[b],
                                    gsem[b])

        def fire_store(c):
            t, j = divmod(c, chunks_per_w)
            b = c % NBUF
            return pltpu.async_copy(
                bufs[b], out_hbm.at[t, pl.ds(base + j * CHUNK, CHUNK)],
                ssem[b])

        gathers = [None] * n_chunks
        stores = [None] * n_chunks
        for c in range(min(NBUF, n_chunks)):
            gathers[c] = fire_gather(c)
        for c in range(n_chunks):
            gathers[c].wait()
            stores[c] = fire_store(c)
            nxt = c + NBUF
            if nxt < n_chunks:
                stores[nxt - NBUF].wait()
                gathers[nxt] = fire_gather(nxt)
        for c in range(max(0, n_chunks - NBUF), n_chunks):
            stores[c].wait()

    return k


def kernel(input_seq, W0, W1, W2):
    b, s = input_seq.shape
    total = b * s
    idx2d = jnp.asarray(input_seq, jnp.int32).reshape(total // CHUNK, CHUNK)
    out = _make_sc_gather(total)(idx2d, W0, W1, W2)
    return out.reshape(NUM_TABLES, b, s, DIM)


# speedup vs baseline: 2.4426x; 1.0080x over previous
import functools

import jax
import jax.numpy as jnp
from jax import lax
from jax.experimental import pallas as pl
from jax.experimental.pallas import tpu as pltpu
from jax.experimental.pallas import tpu_sc as plsc

NUM_TABLES = 3
DIM = 128
CHUNK = 128
NBUF = 6


def _make_sc_gather(total_rows: int):
    info = plsc.get_sparse_core_info()
    nw = info.num_cores * info.num_subcores
    rows_per_w = total_rows // nw
    chunks_per_w = rows_per_w // CHUNK
    idx_rows_per_w = rows_per_w // CHUNK

    mesh = plsc.VectorSubcoreMesh(core_axis_name="c", subcore_axis_name="s")

    @functools.partial(
        pl.kernel,
        out_type=jax.ShapeDtypeStruct((NUM_TABLES, total_rows, DIM),
                                      jnp.float32),
        mesh=mesh,
        scratch_types=(
            [pltpu.VMEM((idx_rows_per_w, CHUNK), jnp.int32)]
            + [pltpu.VMEM((CHUNK, DIM), jnp.float32) for _ in range(NBUF)]
            + [pltpu.SemaphoreType.DMA for _ in range(2 * NBUF)]
        ),
    )
    def k(idx_hbm, w0, w1, w2, out_hbm, idx_v, *bufs_and_sems):
        bufs = bufs_and_sems[:NBUF]
        gsem = bufs_and_sems[NBUF:2 * NBUF]
        ssem = bufs_and_sems[2 * NBUF:]
        wid = lax.axis_index("s") * info.num_cores + lax.axis_index("c")
        base = wid * rows_per_w

        pltpu.sync_copy(idx_hbm.at[pl.ds(wid * idx_rows_per_w,
                                         idx_rows_per_w)], idx_v)

        tables = (w0, w1, w2)
        n_chunks = NUM_TABLES * chunks_per_w

        def fire_gather(c):
            t, j = divmod(c, chunks_per_w)
            b = c % NBUF
            return pltpu.async_copy(tables[t].at[idx_v.at[j]], bufs[b],
                                    gsem[b])

        def fire_store(c):
            t, j = divmod(c, chunks_per_w)
            b = c % NBUF
            return pltpu.async_copy(
                bufs[b], out_hbm.at[t, pl.ds(base + j * CHUNK, CHUNK)],
                ssem[b])

        gathers = [None] * n_chunks
        stores = [None] * n_chunks
        for c in range(min(NBUF, n_chunks)):
            gathers[c] = fire_gather(c)
        for c in range(n_chunks):
            gathers[c].wait()
            stores[c] = fire_store(c)
            nxt = c + NBUF
            if nxt < n_chunks:
                stores[nxt - NBUF].wait()
                gathers[nxt] = fire_gather(nxt)
        for c in range(max(0, n_chunks - NBUF), n_chunks):
            stores[c].wait()

    return k


def kernel(input_seq, W0, W1, W2):
    b, s = input_seq.shape
    total = b * s
    idx2d = jnp.asarray(input_seq, jnp.int32).reshape(total // CHUNK, CHUNK)
    out = _make_sc_gather(total)(idx2d, W0, W1, W2)
    return out.reshape(NUM_TABLES, b, s, DIM)


# speedup vs baseline: 2.4511x; 1.0035x over previous
import functools

import jax
import jax.numpy as jnp
from jax import lax
from jax.experimental import pallas as pl
from jax.experimental.pallas import tpu as pltpu
from jax.experimental.pallas import tpu_sc as plsc

NUM_TABLES = 3
DIM = 128
CHUNK = 128
NBUF = 6


def _make_sc_gather(batch: int, seq: int):
    total_rows = batch * seq
    info = plsc.get_sparse_core_info()
    nw = info.num_cores * info.num_subcores
    rows_per_w = total_rows // nw
    chunks_per_w = rows_per_w // CHUNK
    w_per_row = seq // rows_per_w

    mesh = plsc.VectorSubcoreMesh(core_axis_name="c", subcore_axis_name="s")

    @functools.partial(
        pl.kernel,
        out_type=jax.ShapeDtypeStruct((NUM_TABLES, total_rows, DIM),
                                      jnp.float32),
        mesh=mesh,
        scratch_types=(
            [pltpu.VMEM((rows_per_w,), jnp.int32)]
            + [pltpu.VMEM((CHUNK, DIM), jnp.float32) for _ in range(NBUF)]
            + [pltpu.SemaphoreType.DMA for _ in range(2 * NBUF)]
        ),
    )
    def k(idx_hbm, w0, w1, w2, out_hbm, idx_v, *bufs_and_sems):
        bufs = bufs_and_sems[:NBUF]
        gsem = bufs_and_sems[NBUF:2 * NBUF]
        ssem = bufs_and_sems[2 * NBUF:]
        wid = lax.axis_index("s") * info.num_cores + lax.axis_index("c")
        base = wid * rows_per_w

        pltpu.sync_copy(
            idx_hbm.at[wid // w_per_row,
                       pl.ds((wid % w_per_row) * rows_per_w, rows_per_w)],
            idx_v)

        tables = (w0, w1, w2)
        n_chunks = NUM_TABLES * chunks_per_w

        def fire_gather(c):
            t, j = divmod(c, chunks_per_w)
            b = c % NBUF
            return pltpu.async_copy(
                tables[t].at[idx_v.at[pl.ds(j * CHUNK, CHUNK)]], bufs[b],
                gsem[b])

        def fire_store(c):
            t, j = divmod(c, chunks_per_w)
            b = c % NBUF
            return pltpu.async_copy(
                bufs[b], out_hbm.at[t, pl.ds(base + j * CHUNK, CHUNK)],
                ssem[b])

        gathers = [None] * n_chunks
        stores = [None] * n_chunks
        for c in range(min(NBUF, n_chunks)):
            gathers[c] = fire_gather(c)
        for c in range(n_chunks):
            gathers[c].wait()
            stores[c] = fire_store(c)
            nxt = c + NBUF
            if nxt < n_chunks:
                stores[nxt - NBUF].wait()
                gathers[nxt] = fire_gather(nxt)
        for c in range(max(0, n_chunks - NBUF), n_chunks):
            stores[c].wait()

    return k


def kernel(input_seq, W0, W1, W2):
    b, s = input_seq.shape
    idx = jnp.asarray(input_seq, jnp.int32)
    out = _make_sc_gather(b, s)(idx, W0, W1, W2)
    return out.reshape(NUM_TABLES, b, s, DIM)


# speedup vs baseline: 2.4644x; 1.0054x over previous
import functools

import jax
import jax.numpy as jnp
from jax import lax
from jax.experimental import pallas as pl
from jax.experimental.pallas import tpu as pltpu
from jax.experimental.pallas import tpu_sc as plsc

NUM_TABLES = 3
DIM = 128
CHUNK = 256
NBUF = 3


def _make_sc_gather(batch: int, seq: int):
    total_rows = batch * seq
    info = plsc.get_sparse_core_info()
    nw = info.num_cores * info.num_subcores
    rows_per_w = total_rows // nw
    chunks_per_w = rows_per_w // CHUNK
    w_per_row = seq // rows_per_w

    mesh = plsc.VectorSubcoreMesh(core_axis_name="c", subcore_axis_name="s")

    @functools.partial(
        pl.kernel,
        out_type=jax.ShapeDtypeStruct((NUM_TABLES, total_rows, DIM),
                                      jnp.float32),
        mesh=mesh,
        scratch_types=(
            [pltpu.VMEM((rows_per_w,), jnp.int32)]
            + [pltpu.VMEM((CHUNK, DIM), jnp.float32) for _ in range(NBUF)]
            + [pltpu.SemaphoreType.DMA for _ in range(2 * NBUF)]
        ),
    )
    def k(idx_hbm, w0, w1, w2, out_hbm, idx_v, *bufs_and_sems):
        bufs = bufs_and_sems[:NBUF]
        gsem = bufs_and_sems[NBUF:2 * NBUF]
        ssem = bufs_and_sems[2 * NBUF:]
        wid = lax.axis_index("s") * info.num_cores + lax.axis_index("c")
        base = wid * rows_per_w

        pltpu.sync_copy(
            idx_hbm.at[wid // w_per_row,
                       pl.ds((wid % w_per_row) * rows_per_w, rows_per_w)],
            idx_v)

        tables = (w0, w1, w2)
        n_chunks = NUM_TABLES * chunks_per_w

        def fire_gather(c):
            t, j = divmod(c, chunks_per_w)
            b = c % NBUF
            return pltpu.async_copy(
                tables[t].at[idx_v.at[pl.ds(j * CHUNK, CHUNK)]], bufs[b],
                gsem[b])

        def fire_store(c):
            t, j = divmod(c, chunks_per_w)
            b = c % NBUF
            return pltpu.async_copy(
                bufs[b], out_hbm.at[t, pl.ds(base + j * CHUNK, CHUNK)],
                ssem[b])

        gathers = [None] * n_chunks
        stores = [None] * n_chunks
        for c in range(min(NBUF, n_chunks)):
            gathers[c] = fire_gather(c)
        for c in range(n_chunks):
            gathers[c].wait()
            stores[c] = fire_store(c)
            nxt = c + NBUF
            if nxt < n_chunks:
                stores[nxt - NBUF].wait()
                gathers[nxt] = fire_gather(nxt)
        for c in range(max(0, n_chunks - NBUF), n_chunks):
            stores[c].wait()

    return k


def kernel(input_seq, W0, W1, W2):
    b, s = input_seq.shape
    idx = jnp.asarray(input_seq, jnp.int32)
    out = _make_sc_gather(b, s)(idx, W0, W1, W2)
    return out.reshape(NUM_TABLES, b, s, DIM)
